# Initial kernel scaffold; baseline (speedup 1.0000x reference)
import jax
import jax.numpy as jnp
from jax.experimental import pallas as pl


def kernel(x):
    raise NotImplementedError("write your pallas kernel here")


# speedup vs baseline: 27.0720x; 27.0720x over previous
import jax
import jax.numpy as jnp
from jax.experimental import pallas as pl

N = 5000
R, C = 40, 128
NP = R * C
K = 100
NEG = -1e30
IOU_T = 0.25
CONF_T = 0.5
IMG = 512.0


def _post_body(x_ref, o_ref):
    col = lambda c: x_ref[c]

    conf = col(0)
    clsf = jnp.zeros((R, C), jnp.float32)
    for c in range(1, 10):
        sc_c = col(c)
        take = sc_c > conf
        conf = jnp.where(take, sc_c, conf)
        clsf = jnp.where(take, jnp.float32(c), clsf)

    c10, c11, c12, c13 = col(10), col(11), col(12), col(13)
    c14, c15, c16, c17 = col(14), col(15), col(16), col(17)
    c18, c19, c20, c21 = col(18), col(19), col(20), col(21)

    pred41 = jnp.exp(c12 * c20)
    pred51 = jnp.exp(c13 * c21)
    w = pred41 + pred41 * c16
    h = pred51 + pred51 * c17
    cx = c14 + c10 * c18 * c16
    cy = c15 + c11 * c19 * c17
    x1v = jnp.maximum(cx - w / 2.0, 0.0)
    x2v = jnp.minimum(cx + w / 2.0, IMG - 1.0)
    y1v = jnp.maximum(cy - h / 2.0, 0.0)
    y2v = jnp.minimum(cy + h / 2.0, IMG - 1.0)
    areas = (y2v - y1v) * (x2v - x1v)

    giota = (jax.lax.broadcasted_iota(jnp.int32, (R, C), 0) * C
             + jax.lax.broadcasted_iota(jnp.int32, (R, C), 1))
    inb = giota < N
    valid = (clsf != 0.0) & ((conf - CONF_T) != 0.0) & inb
    s0 = jnp.where(valid, conf, NEG)

    lane_i = jax.lax.broadcasted_iota(jnp.int32, (1, C), 1)
    sub_i = jax.lax.broadcasted_iota(jnp.int32, (C, 1), 0)
    zrow = jnp.zeros((1, C), jnp.float32)
    zcol = jnp.zeros((C, 1), jnp.float32)

    def body(k, carry):
        s, cls_r, sc_r, y1_r, x1_r, y2_r, x2_r, ok_r, cls_c, ok_c = carry
        m = jnp.max(s)
        idx = jnp.min(jnp.where(s == m, giota, jnp.int32(2 ** 30)))
        ok = m > (NEG / 2)
        onehot = giota == idx
        sel = lambda v: jnp.sum(jnp.where(onehot, v, 0.0))
        y1i = sel(y1v)
        x1i = sel(x1v)
        y2i = sel(y2v)
        x2i = sel(x2v)
        ci = sel(clsf)
        ai = sel(areas)
        yy1 = jnp.maximum(y1i, y1v)
        xx1 = jnp.maximum(x1i, x1v)
        yy2 = jnp.minimum(y2i, y2v)
        xx2 = jnp.minimum(x2i, x2v)
        inter = jnp.maximum(yy2 - yy1, 0.0) * jnp.maximum(xx2 - xx1, 0.0)
        iou = inter / (ai + areas - inter + 1e-12)
        s = jnp.where(ok & (iou > IOU_T), NEG, s)
        s = jnp.where(onehot, NEG, s)

        okf = jnp.where(ok, 1.0, 0.0)
        lm = lane_i == k
        cm = sub_i == k
        cls_r = jnp.where(lm, ci, cls_r)
        sc_r = jnp.where(lm, m, sc_r)
        y1_r = jnp.where(lm, y1i, y1_r)
        x1_r = jnp.where(lm, x1i, x1_r)
        y2_r = jnp.where(lm, y2i, y2_r)
        x2_r = jnp.where(lm, x2i, x2_r)
        ok_r = jnp.where(lm, okf, ok_r)
        cls_c = jnp.where(cm, ci, cls_c)
        ok_c = jnp.where(cm, okf, ok_c)
        return (s, cls_r, sc_r, y1_r, x1_r, y2_r, x2_r, ok_r, cls_c, ok_c)

    carry = (s0, zrow, zrow, zrow, zrow, zrow, zrow, zrow, zcol, zcol)
    (s, cls_r, sc_r, y1_r, x1_r, y2_r, x2_r, ok_r,
     cls_c, ok_c) = jax.lax.fori_loop(0, K, body, carry)

    okb_r = ok_r > 0.5
    okb_c = ok_c > 0.5
    key_r = jnp.where(okb_r, cls_r, 1e6)
    key_r = jnp.where(lane_i < K, key_r, 2e6)
    key_c = jnp.where(okb_c, cls_c, 1e6)
    key_c = jnp.where(sub_i < K, key_c, 2e6)

    kr = jnp.broadcast_to(key_r, (C, C))
    kc = jnp.broadcast_to(key_c, (C, C))
    sub2 = jax.lax.broadcasted_iota(jnp.int32, (C, C), 0)
    lane2 = jax.lax.broadcasted_iota(jnp.int32, (C, C), 1)
    lt = (kc < kr).astype(jnp.float32)
    eq = ((kc == kr) & (sub2 < lane2)).astype(jnp.float32)
    rank_r = jnp.sum(lt + eq, axis=0, keepdims=True)

    perm = (jnp.broadcast_to(rank_r, (C, C))
            == sub2.astype(jnp.float32)).astype(jnp.float32)

    vals = (
        jnp.where(okb_r, cls_r, 0.0),
        jnp.where(okb_r, sc_r, 0.0),
        jnp.where(okb_r, y1_r / IMG, 0.0),
        jnp.where(okb_r, x1_r / IMG, 0.0),
        jnp.where(okb_r, 1.0 - y2_r / IMG, 0.0),
        jnp.where(okb_r, 1.0 - x2_r / IMG, 0.0),
    )
    cols = [jnp.sum(perm * jnp.broadcast_to(v, (C, C)), axis=1, keepdims=True)
            for v in vals]
    res = jnp.concatenate(cols, axis=1)
    o_ref[...] = res[:K, :]


def kernel(x):
    x0 = jnp.transpose(x[0])
    x22 = jnp.pad(x0, ((0, 0), (0, NP - N))).reshape(22, R, C)
    out = pl.pallas_call(
        _post_body,
        out_shape=jax.ShapeDtypeStruct((K, 6), jnp.float32),
    )(x22)
    return jnp.broadcast_to(out[None], (x.shape[0], K, 6))


# speedup vs baseline: 167.8275x; 6.1993x over previous
import jax
import jax.numpy as jnp
from jax.experimental import pallas as pl
from jax.experimental.pallas import tpu as pltpu

N = 5000
R, C = 40, 128
NP = R * C
K = 100
NEG = -1e30
IOU_T = 0.25
CONF_T = 0.5
IMG = 512.0


def _post_body(x_ref, o_ref, box_ref):
    col = lambda c: x_ref[c]

    conf = col(0)
    clsf = jnp.zeros((R, C), jnp.float32)
    for c in range(1, 10):
        sc_c = col(c)
        take = sc_c > conf
        conf = jnp.where(take, sc_c, conf)
        clsf = jnp.where(take, jnp.float32(c), clsf)

    c10, c11, c12, c13 = col(10), col(11), col(12), col(13)
    c14, c15, c16, c17 = col(14), col(15), col(16), col(17)
    c18, c19, c20, c21 = col(18), col(19), col(20), col(21)

    pred41 = jnp.exp(c12 * c20)
    pred51 = jnp.exp(c13 * c21)
    w = pred41 + pred41 * c16
    h = pred51 + pred51 * c17
    cx = c14 + c10 * c18 * c16
    cy = c15 + c11 * c19 * c17
    x1v = jnp.maximum(cx - w / 2.0, 0.0)
    x2v = jnp.minimum(cx + w / 2.0, IMG - 1.0)
    y1v = jnp.maximum(cy - h / 2.0, 0.0)
    y2v = jnp.minimum(cy + h / 2.0, IMG - 1.0)
    areas = (y2v - y1v) * (x2v - x1v)

    box_ref[:, 0, :] = y1v
    box_ref[:, 1, :] = x1v
    box_ref[:, 2, :] = y2v
    box_ref[:, 3, :] = x2v
    box_ref[:, 4, :] = clsf
    box_ref[:, 5, :] = areas

    giota = (jax.lax.broadcasted_iota(jnp.int32, (R, C), 0) * C
             + jax.lax.broadcasted_iota(jnp.int32, (R, C), 1))
    inb = giota < N
    valid = (clsf != 0.0) & ((conf - CONF_T) != 0.0) & inb
    s0 = jnp.where(valid, conf, NEG)

    lane_i = jax.lax.broadcasted_iota(jnp.int32, (1, C), 1)
    sub_i = jax.lax.broadcasted_iota(jnp.int32, (C, 1), 0)
    zrow = jnp.zeros((1, C), jnp.float32)
    zcol = jnp.zeros((C, 1), jnp.float32)

    def cond(carry):
        k, ok, *_ = carry
        return (k < K) & ok

    def body(carry):
        k, _, s, cls_r, sc_r, y1_r, x1_r, y2_r, x2_r, ok_r, cls_c, ok_c = carry
        m = jnp.max(s)
        idx = jnp.min(jnp.where(s == m, giota, jnp.int32(2 ** 30)))
        ok = m > (NEG / 2)
        row = idx // C
        lane = idx - row * C
        rowmat = box_ref[pl.ds(row, 1), :, :].reshape(8, C)
        sums = jnp.sum(jnp.where(lane_i == lane, rowmat, 0.0),
                       axis=1, keepdims=True)
        y1i = sums[0:1, :]
        x1i = sums[1:2, :]
        y2i = sums[2:3, :]
        x2i = sums[3:4, :]
        ci = sums[4:5, :]
        ai = sums[5:6, :]
        yy1 = jnp.maximum(y1i, y1v)
        xx1 = jnp.maximum(x1i, x1v)
        yy2 = jnp.minimum(y2i, y2v)
        xx2 = jnp.minimum(x2i, x2v)
        inter = jnp.maximum(yy2 - yy1, 0.0) * jnp.maximum(xx2 - xx1, 0.0)
        iou = inter / (ai + areas - inter + 1e-12)
        s = jnp.where(ok & (iou > IOU_T), NEG, s)
        s = jnp.where(giota == idx, NEG, s)

        okf = jnp.where(ok, 1.0, 0.0)
        lm = lane_i == k
        cm = sub_i == k
        cls_r = jnp.where(lm, ci, cls_r)
        sc_r = jnp.where(lm, m, sc_r)
        y1_r = jnp.where(lm, y1i, y1_r)
        x1_r = jnp.where(lm, x1i, x1_r)
        y2_r = jnp.where(lm, y2i, y2_r)
        x2_r = jnp.where(lm, x2i, x2_r)
        ok_r = jnp.where(lm, okf, ok_r)
        cls_c = jnp.where(cm, ci, cls_c)
        ok_c = jnp.where(cm, okf, ok_c)
        return (k + 1, ok, s, cls_r, sc_r, y1_r, x1_r, y2_r, x2_r,
                ok_r, cls_c, ok_c)

    carry = (jnp.int32(0), jnp.bool_(True), s0, zrow, zrow, zrow, zrow,
             zrow, zrow, zrow, zcol, zcol)
    (_, _, s, cls_r, sc_r, y1_r, x1_r, y2_r, x2_r, ok_r,
     cls_c, ok_c) = jax.lax.while_loop(cond, body, carry)

    okb_r = ok_r > 0.5
    okb_c = ok_c > 0.5
    key_r = jnp.where(okb_r, cls_r, 1e6)
    key_r = jnp.where(lane_i < K, key_r, 2e6)
    key_c = jnp.where(okb_c, cls_c, 1e6)
    key_c = jnp.where(sub_i < K, key_c, 2e6)

    kr = jnp.broadcast_to(key_r, (C, C))
    kc = jnp.broadcast_to(key_c, (C, C))
    sub2 = jax.lax.broadcasted_iota(jnp.int32, (C, C), 0)
    lane2 = jax.lax.broadcasted_iota(jnp.int32, (C, C), 1)
    lt = (kc < kr).astype(jnp.float32)
    eq = ((kc == kr) & (sub2 < lane2)).astype(jnp.float32)
    rank_r = jnp.sum(lt + eq, axis=0, keepdims=True)

    perm = (jnp.broadcast_to(rank_r, (C, C))
            == sub2.astype(jnp.float32)).astype(jnp.float32)

    vals = (
        jnp.where(okb_r, cls_r, 0.0),
        jnp.where(okb_r, sc_r, 0.0),
        jnp.where(okb_r, y1_r / IMG, 0.0),
        jnp.where(okb_r, x1_r / IMG, 0.0),
        jnp.where(okb_r, 1.0 - y2_r / IMG, 0.0),
        jnp.where(okb_r, 1.0 - x2_r / IMG, 0.0),
    )
    cols = [jnp.sum(perm * jnp.broadcast_to(v, (C, C)), axis=1, keepdims=True)
            for v in vals]
    res = jnp.concatenate(cols, axis=1)
    o_ref[...] = res[:K, :]


def kernel(x):
    x0 = jnp.transpose(x[0])
    x22 = jnp.pad(x0, ((0, 0), (0, NP - N))).reshape(22, R, C)
    out = pl.pallas_call(
        _post_body,
        out_shape=jax.ShapeDtypeStruct((K, 6), jnp.float32),
        scratch_shapes=[pltpu.VMEM((R, 8, C), jnp.float32)],
    )(x22)
    return jnp.broadcast_to(out[None], (x.shape[0], K, 6))
